# Initial kernel scaffold; baseline (speedup 1.0000x reference)
import jax
import jax.numpy as jnp
from jax.experimental import pallas as pl


def kernel(x, edge_index, W1, b1, g1, be1, rm1, rv1, W2, b2, g2, be2, rm2, rv2, W3, b3):
    raise NotImplementedError("write your pallas kernel here")


# speedup vs baseline: 6.3803x; 6.3803x over previous
import functools

import jax
import jax.numpy as jnp
from jax import lax
from jax.experimental import pallas as pl
from jax.experimental.pallas import tpu as pltpu
from jax.experimental.pallas import tpu_sc as plsc

_N = 10000
_D = 128
_E = 320000
_NC, _NS = 2, 16
_NW = _NC * _NS
_NP = 10240
_ROWS_PT = _NP // _NS
_EP = 327680
_EPT = _EP // _NW
_NCHUNK = _EPT // 128

_MESH = plsc.VectorSubcoreMesh(core_axis_name="c", subcore_axis_name="s")


@functools.partial(
    pl.kernel,
    mesh=_MESH,
    out_type=jax.ShapeDtypeStruct((_NC, _NP), jnp.float32),
    scratch_types=[
        pltpu.VMEM((_NCHUNK, 128), jnp.int32),
        pltpu.VMEM((128,), jnp.float32),
        pltpu.VMEM_SHARED((_NP,), jnp.float32),
    ],
)
def _deg(dst_hbm, out_hbm, dst_v, ones_v, acc):
    c = lax.axis_index("c")
    s = lax.axis_index("s")
    wid = c * _NS + s

    for j in range(8):
        ones_v[pl.ds(j * 16, 16)] = jnp.zeros((16,), jnp.float32)
    for k in range(_ROWS_PT // 128):
        pltpu.sync_copy(ones_v, acc.at[pl.ds(s * _ROWS_PT + k * 128, 128)])
    for j in range(8):
        ones_v[pl.ds(j * 16, 16)] = jnp.ones((16,), jnp.float32)
    plsc.subcore_barrier()

    pltpu.sync_copy(dst_hbm.at[pl.ds(wid * _NCHUNK, _NCHUNK)], dst_v)

    def body(i, _):
        pltpu.sync_copy(ones_v, acc.at[dst_v.at[i]], add=True)
        return 0
    lax.fori_loop(0, _NCHUNK, body, 0)

    plsc.subcore_barrier()
    pltpu.sync_copy(acc.at[pl.ds(s * _ROWS_PT, _ROWS_PT)],
                    out_hbm.at[c, pl.ds(s * _ROWS_PT, _ROWS_PT)])


@functools.partial(
    pl.kernel,
    mesh=_MESH,
    out_type=jax.ShapeDtypeStruct((_NC, _NP, _D), jnp.float32),
    scratch_types=[
        pltpu.VMEM((_NCHUNK, 128), jnp.int32),
        pltpu.VMEM((_NCHUNK, 128), jnp.int32),
        pltpu.VMEM((128, _D), jnp.float32),
        pltpu.VMEM_SHARED((_NP, _D), jnp.float32),
        pltpu.SemaphoreType.DMA,
    ],
)
def _agg(y_hbm, src_hbm, dst_hbm, out_hbm, src_v, dst_v, rows_v, acc, sem):
    c = lax.axis_index("c")
    s = lax.axis_index("s")
    wid = c * _NS + s

    def zero_rows(i, _):
        for j in range(_D // 16):
            rows_v[i, pl.ds(j * 16, 16)] = jnp.zeros((16,), jnp.float32)
        return 0
    lax.fori_loop(0, 128, zero_rows, 0)
    for k in range(_ROWS_PT // 128):
        pltpu.sync_copy(rows_v, acc.at[pl.ds(s * _ROWS_PT + k * 128, 128)])
    plsc.subcore_barrier()

    pltpu.sync_copy(src_hbm.at[pl.ds(wid * _NCHUNK, _NCHUNK)], src_v)
    pltpu.sync_copy(dst_hbm.at[pl.ds(wid * _NCHUNK, _NCHUNK)], dst_v)

    def body(i, _):
        pltpu.async_copy(y_hbm.at[src_v.at[i]], rows_v, sem).wait()
        pltpu.sync_copy(rows_v, acc.at[dst_v.at[i]], add=True)
        return 0
    lax.fori_loop(0, _NCHUNK, body, 0)

    plsc.subcore_barrier()
    pltpu.sync_copy(acc.at[pl.ds(s * _ROWS_PT, _ROWS_PT)],
                    out_hbm.at[c, pl.ds(s * _ROWS_PT, _ROWS_PT)])


_R = 256
_GRID = _NP // _R


def _dk_body(p_ref, o_ref):
    o_ref[...] = lax.rsqrt(p_ref[0] + p_ref[1] + 1.0)


_dk = pl.pallas_call(
    _dk_body,
    in_specs=[pl.BlockSpec((_NC, _NP // 128, 128), lambda: (0, 0, 0))],
    out_specs=pl.BlockSpec((_NP // 128, 128), lambda: (0, 0)),
    out_shape=jax.ShapeDtypeStruct((_NP // 128, 128), jnp.float32),
)


def _mm1_body(x_ref, w_ref, d_ref, o_ref):
    xw = jnp.dot(x_ref[...], w_ref[...], preferred_element_type=jnp.float32)
    o_ref[...] = d_ref[...] * xw


_mm1 = pl.pallas_call(
    _mm1_body,
    grid=(_GRID,),
    in_specs=[
        pl.BlockSpec((_R, _D), lambda i: (i, 0)),
        pl.BlockSpec((_D, _D), lambda i: (0, 0)),
        pl.BlockSpec((_R, 1), lambda i: (i, 0)),
    ],
    out_specs=pl.BlockSpec((_R, _D), lambda i: (i, 0)),
    out_shape=jax.ShapeDtypeStruct((_NP, _D), jnp.float32),
)


def _mid_body(e0_ref, e1_ref, y_ref, d_ref, w_ref, s_ref, t_ref, b_ref, o_ref):
    pre = d_ref[...] * (e0_ref[...] + e1_ref[...] + y_ref[...]) + b_ref[...]
    h = jnp.maximum(s_ref[...] * pre + t_ref[...], 0.0)
    o_ref[...] = d_ref[...] * jnp.dot(
        h, w_ref[...], preferred_element_type=jnp.float32)


_mid = pl.pallas_call(
    _mid_body,
    grid=(_GRID,),
    in_specs=[
        pl.BlockSpec((_R, _D), lambda i: (i, 0)),
        pl.BlockSpec((_R, _D), lambda i: (i, 0)),
        pl.BlockSpec((_R, _D), lambda i: (i, 0)),
        pl.BlockSpec((_R, 1), lambda i: (i, 0)),
        pl.BlockSpec((_D, _D), lambda i: (0, 0)),
        pl.BlockSpec((1, _D), lambda i: (0, 0)),
        pl.BlockSpec((1, _D), lambda i: (0, 0)),
        pl.BlockSpec((1, _D), lambda i: (0, 0)),
    ],
    out_specs=pl.BlockSpec((_R, _D), lambda i: (i, 0)),
    out_shape=jax.ShapeDtypeStruct((_NP, _D), jnp.float32),
)


def _fin_body(e0_ref, e1_ref, y_ref, d_ref, b_ref, o_ref):
    u = d_ref[...] * (e0_ref[...] + e1_ref[...] + y_ref[...]) + b_ref[...]
    nrm = jnp.sqrt(jnp.sum(u * u, axis=1, keepdims=True))
    o_ref[...] = u / jnp.maximum(nrm, 1e-12)


_fin = pl.pallas_call(
    _fin_body,
    grid=(_GRID,),
    in_specs=[
        pl.BlockSpec((_R, _D), lambda i: (i, 0)),
        pl.BlockSpec((_R, _D), lambda i: (i, 0)),
        pl.BlockSpec((_R, _D), lambda i: (i, 0)),
        pl.BlockSpec((_R, 1), lambda i: (i, 0)),
        pl.BlockSpec((1, _D), lambda i: (0, 0)),
    ],
    out_specs=pl.BlockSpec((_R, _D), lambda i: (i, 0)),
    out_shape=jax.ShapeDtypeStruct((_NP, _D), jnp.float32),
)


def kernel(x, edge_index, W1, b1, g1, be1, rm1, rv1,
           W2, b2, g2, be2, rm2, rv2, W3, b3):
    f32 = jnp.float32
    eps = 1e-5

    x_p = jnp.zeros((_NP, _D), f32).at[:_N].set(x)
    pad = jnp.full((_EP - _E,), _N, jnp.int32)
    src2d = jnp.concatenate([edge_index[0], pad]).reshape(_EP // 128, 128)
    dst2d = jnp.concatenate([edge_index[1], pad]).reshape(_EP // 128, 128)

    pdeg = _deg(dst2d)
    dmat = _dk(pdeg.reshape(_NC, _NP // 128, 128))
    d_col = dmat.reshape(_NP, 1)

    s1 = (g1 / jnp.sqrt(rv1 + eps)).reshape(1, _D)
    t1 = (be1 - rm1 * (g1 / jnp.sqrt(rv1 + eps))).reshape(1, _D)
    s2 = (g2 / jnp.sqrt(rv2 + eps)).reshape(1, _D)
    t2 = (be2 - rm2 * (g2 / jnp.sqrt(rv2 + eps))).reshape(1, _D)

    y1 = _mm1(x_p, W1, d_col)
    a1 = _agg(y1, src2d, dst2d)
    y2 = _mid(a1[0], a1[1], y1, d_col, W2, s1, t1, b1.reshape(1, _D))
    a2 = _agg(y2, src2d, dst2d)
    y3 = _mid(a2[0], a2[1], y2, d_col, W3, s2, t2, b2.reshape(1, _D))
    a3 = _agg(y3, src2d, dst2d)
    out = _fin(a3[0], a3[1], y3, d_col, b3.reshape(1, _D))
    return out[:_N]


# speedup vs baseline: 6.8865x; 1.0793x over previous
import functools

import jax
import jax.numpy as jnp
from jax import lax
from jax.experimental import pallas as pl
from jax.experimental.pallas import tpu as pltpu
from jax.experimental.pallas import tpu_sc as plsc

_N = 10000
_D = 128
_E = 320000
_NC, _NS = 2, 16
_NW = _NC * _NS
_NP = 10240
_ROWS_PT = _NP // _NS
_EP = 327680
_EPT = _EP // _NW
_NCHUNK = _EPT // 128

_MESH = plsc.VectorSubcoreMesh(core_axis_name="c", subcore_axis_name="s")


@functools.partial(
    pl.kernel,
    mesh=_MESH,
    out_type=jax.ShapeDtypeStruct((_NC, _NP), jnp.float32),
    scratch_types=[
        pltpu.VMEM((_NCHUNK, 128), jnp.int32),
        pltpu.VMEM((128,), jnp.float32),
        pltpu.VMEM_SHARED((_NP,), jnp.float32),
    ],
)
def _deg(dst_hbm, out_hbm, dst_v, ones_v, acc):
    c = lax.axis_index("c")
    s = lax.axis_index("s")
    wid = c * _NS + s

    for j in range(8):
        ones_v[pl.ds(j * 16, 16)] = jnp.zeros((16,), jnp.float32)
    for k in range(_ROWS_PT // 128):
        pltpu.sync_copy(ones_v, acc.at[pl.ds(s * _ROWS_PT + k * 128, 128)])
    for j in range(8):
        ones_v[pl.ds(j * 16, 16)] = jnp.ones((16,), jnp.float32)
    plsc.subcore_barrier()

    pltpu.sync_copy(dst_hbm.at[pl.ds(wid * _NCHUNK, _NCHUNK)], dst_v)

    def body(i, _):
        pltpu.sync_copy(ones_v, acc.at[dst_v.at[i]], add=True)
        return 0
    lax.fori_loop(0, _NCHUNK, body, 0)

    plsc.subcore_barrier()
    pltpu.sync_copy(acc.at[pl.ds(s * _ROWS_PT, _ROWS_PT)],
                    out_hbm.at[c, pl.ds(s * _ROWS_PT, _ROWS_PT)])


_NBUF = 2
_GRP = 8
_HALF = _NCHUNK // 2


@functools.partial(
    pl.kernel,
    mesh=_MESH,
    out_type=jax.ShapeDtypeStruct((_NC, _NP, _D), jnp.float32),
    scratch_types=[
        pltpu.VMEM((_HALF, 128), jnp.int32),
        pltpu.VMEM((_HALF, 128), jnp.int32),
    ] + [pltpu.VMEM((128, _D), jnp.float32) for _ in range(_NBUF)]
      + [pltpu.VMEM_SHARED((_NP, _D), jnp.float32)]
      + [pltpu.SemaphoreType.DMA for _ in range(_NBUF)],
)
def _agg(y_hbm, src_hbm, dst_hbm, out_hbm, src_v, dst_v,
         r0, r1, acc, g0, g1):
    rows = (r0, r1)
    gsems = (g0, g1)
    c = lax.axis_index("c")
    s = lax.axis_index("s")
    wid = c * _NS + s

    def zero_rows(i, _):
        for j in range(_D // 16):
            r0[i, pl.ds(j * 16, 16)] = jnp.zeros((16,), jnp.float32)
        return 0
    lax.fori_loop(0, 128, zero_rows, 0)
    for k in range(_ROWS_PT // 128):
        pltpu.sync_copy(r0, acc.at[pl.ds(s * _ROWS_PT + k * 128, 128)])
    plsc.subcore_barrier()

    def outer(k, _):
        base = k * _GRP
        hs = [None] * _GRP
        for b in range(_NBUF):
            hs[b] = pltpu.async_copy(
                y_hbm.at[src_v.at[base + b]], rows[b], gsems[b])
        for b in range(_GRP):
            hs[b].wait()
            pltpu.sync_copy(rows[b % _NBUF], acc.at[dst_v.at[base + b]],
                            add=True)
            if b + _NBUF < _GRP:
                hs[b + _NBUF] = pltpu.async_copy(
                    y_hbm.at[src_v.at[base + b + _NBUF]],
                    rows[b % _NBUF], gsems[b % _NBUF])
        return 0

    for half in range(2):
        pltpu.sync_copy(
            src_hbm.at[pl.ds(wid * _NCHUNK + half * _HALF, _HALF)], src_v)
        pltpu.sync_copy(
            dst_hbm.at[pl.ds(wid * _NCHUNK + half * _HALF, _HALF)], dst_v)
        lax.fori_loop(0, _HALF // _GRP, outer, 0)

    plsc.subcore_barrier()
    pltpu.sync_copy(acc.at[pl.ds(s * _ROWS_PT, _ROWS_PT)],
                    out_hbm.at[c, pl.ds(s * _ROWS_PT, _ROWS_PT)])


_R = 256
_GRID = _NP // _R


def _dk_body(p_ref, o_ref):
    o_ref[...] = lax.rsqrt(p_ref[0] + p_ref[1] + 1.0)


_dk = pl.pallas_call(
    _dk_body,
    in_specs=[pl.BlockSpec((_NC, _NP // 128, 128), lambda: (0, 0, 0))],
    out_specs=pl.BlockSpec((_NP // 128, 128), lambda: (0, 0)),
    out_shape=jax.ShapeDtypeStruct((_NP // 128, 128), jnp.float32),
)


def _mm1_body(x_ref, w_ref, d_ref, o_ref):
    xw = jnp.dot(x_ref[...], w_ref[...], preferred_element_type=jnp.float32)
    o_ref[...] = d_ref[...] * xw


_mm1 = pl.pallas_call(
    _mm1_body,
    grid=(_GRID,),
    in_specs=[
        pl.BlockSpec((_R, _D), lambda i: (i, 0)),
        pl.BlockSpec((_D, _D), lambda i: (0, 0)),
        pl.BlockSpec((_R, 1), lambda i: (i, 0)),
    ],
    out_specs=pl.BlockSpec((_R, _D), lambda i: (i, 0)),
    out_shape=jax.ShapeDtypeStruct((_NP, _D), jnp.float32),
)


def _mid_body(e0_ref, e1_ref, y_ref, d_ref, w_ref, s_ref, t_ref, b_ref, o_ref):
    pre = d_ref[...] * (e0_ref[...] + e1_ref[...] + y_ref[...]) + b_ref[...]
    h = jnp.maximum(s_ref[...] * pre + t_ref[...], 0.0)
    o_ref[...] = d_ref[...] * jnp.dot(
        h, w_ref[...], preferred_element_type=jnp.float32)


_mid = pl.pallas_call(
    _mid_body,
    grid=(_GRID,),
    in_specs=[
        pl.BlockSpec((_R, _D), lambda i: (i, 0)),
        pl.BlockSpec((_R, _D), lambda i: (i, 0)),
        pl.BlockSpec((_R, _D), lambda i: (i, 0)),
        pl.BlockSpec((_R, 1), lambda i: (i, 0)),
        pl.BlockSpec((_D, _D), lambda i: (0, 0)),
        pl.BlockSpec((1, _D), lambda i: (0, 0)),
        pl.BlockSpec((1, _D), lambda i: (0, 0)),
        pl.BlockSpec((1, _D), lambda i: (0, 0)),
    ],
    out_specs=pl.BlockSpec((_R, _D), lambda i: (i, 0)),
    out_shape=jax.ShapeDtypeStruct((_NP, _D), jnp.float32),
)


def _fin_body(e0_ref, e1_ref, y_ref, d_ref, b_ref, o_ref):
    u = d_ref[...] * (e0_ref[...] + e1_ref[...] + y_ref[...]) + b_ref[...]
    nrm = jnp.sqrt(jnp.sum(u * u, axis=1, keepdims=True))
    o_ref[...] = u / jnp.maximum(nrm, 1e-12)


_fin = pl.pallas_call(
    _fin_body,
    grid=(_GRID,),
    in_specs=[
        pl.BlockSpec((_R, _D), lambda i: (i, 0)),
        pl.BlockSpec((_R, _D), lambda i: (i, 0)),
        pl.BlockSpec((_R, _D), lambda i: (i, 0)),
        pl.BlockSpec((_R, 1), lambda i: (i, 0)),
        pl.BlockSpec((1, _D), lambda i: (0, 0)),
    ],
    out_specs=pl.BlockSpec((_R, _D), lambda i: (i, 0)),
    out_shape=jax.ShapeDtypeStruct((_NP, _D), jnp.float32),
)


def kernel(x, edge_index, W1, b1, g1, be1, rm1, rv1,
           W2, b2, g2, be2, rm2, rv2, W3, b3):
    f32 = jnp.float32
    eps = 1e-5

    x_p = jnp.zeros((_NP, _D), f32).at[:_N].set(x)
    pad = jnp.full((_EP - _E,), _N, jnp.int32)
    src2d = jnp.concatenate([edge_index[0], pad]).reshape(_EP // 128, 128)
    dst2d = jnp.concatenate([edge_index[1], pad]).reshape(_EP // 128, 128)

    pdeg = _deg(dst2d)
    dmat = _dk(pdeg.reshape(_NC, _NP // 128, 128))
    d_col = dmat.reshape(_NP, 1)

    s1 = (g1 / jnp.sqrt(rv1 + eps)).reshape(1, _D)
    t1 = (be1 - rm1 * (g1 / jnp.sqrt(rv1 + eps))).reshape(1, _D)
    s2 = (g2 / jnp.sqrt(rv2 + eps)).reshape(1, _D)
    t2 = (be2 - rm2 * (g2 / jnp.sqrt(rv2 + eps))).reshape(1, _D)

    y1 = _mm1(x_p, W1, d_col)
    a1 = _agg(y1, src2d, dst2d)
    y2 = _mid(a1[0], a1[1], y1, d_col, W2, s1, t1, b1.reshape(1, _D))
    a2 = _agg(y2, src2d, dst2d)
    y3 = _mid(a2[0], a2[1], y2, d_col, W3, s2, t2, b2.reshape(1, _D))
    a3 = _agg(y3, src2d, dst2d)
    out = _fin(a3[0], a3[1], y3, d_col, b3.reshape(1, _D))
    return out[:_N]
